# Initial kernel scaffold; baseline (speedup 1.0000x reference)
import jax
import jax.numpy as jnp
from jax.experimental import pallas as pl


def kernel(x, edge_index, edge_attr, batch, params):
    raise NotImplementedError("write your pallas kernel here")


# speedup vs baseline: 6.1847x; 6.1847x over previous
import jax
import jax.numpy as jnp
import jax.lax as lax
from jax.experimental import pallas as pl

N_REAL = 50000
E_REAL = 800000
G = 64
NB = 512
EB = 2048
N_PAD = 50176
E_PAD = 800768
H8 = 8


def _proj_kernel(x_ref, wl_ref, wr_ref, xl_ref, xr_ref):
    x = x_ref[...]
    xl_ref[...] = jnp.dot(x, wl_ref[...], preferred_element_type=jnp.float32)
    xr_ref[...] = jnp.dot(x, wr_ref[...], preferred_element_type=jnp.float32)


def _proj(x, wl, wr):
    n, f = x.shape
    hc = wl.shape[1]
    return pl.pallas_call(
        _proj_kernel,
        grid=(n // NB,),
        in_specs=[
            pl.BlockSpec((NB, f), lambda i: (i, 0)),
            pl.BlockSpec((f, hc), lambda i: (0, 0)),
            pl.BlockSpec((f, hc), lambda i: (0, 0)),
        ],
        out_specs=[
            pl.BlockSpec((NB, hc), lambda i: (i, 0)),
            pl.BlockSpec((NB, hc), lambda i: (i, 0)),
        ],
        out_shape=[jax.ShapeDtypeStruct((n, hc), jnp.float32)] * 2,
    )(x, wl, wr)


def _logits_kernel(xd_ref, xs_ref, ea_ref, we_ref, att_ref, lg_ref):
    z = xd_ref[...] + xs_ref[...] + jnp.dot(
        ea_ref[...], we_ref[...], preferred_element_type=jnp.float32)
    g = jnp.where(z >= 0, z, 0.2 * z)
    lg_ref[...] = jnp.dot(g, att_ref[...], preferred_element_type=jnp.float32)


def _logits(xld, xrs, eap, we8, att_bd):
    hc = xld.shape[1]
    return pl.pallas_call(
        _logits_kernel,
        grid=(E_PAD // EB,),
        in_specs=[
            pl.BlockSpec((EB, hc), lambda i: (i, 0)),
            pl.BlockSpec((EB, hc), lambda i: (i, 0)),
            pl.BlockSpec((EB, 8), lambda i: (i, 0)),
            pl.BlockSpec((8, hc), lambda i: (0, 0)),
            pl.BlockSpec((hc, H8), lambda i: (0, 0)),
        ],
        out_specs=pl.BlockSpec((EB, H8), lambda i: (i, 0)),
        out_shape=jax.ShapeDtypeStruct((E_PAD, H8), jnp.float32),
    )(xld, xrs, eap, we8, att_bd)


def _msg_kernel(lg_ref, m_ref, exp_ref, xs_ref, wm_ref, ex_ref):
    ex = jnp.exp(lg_ref[...] - m_ref[0:1, :])
    ex_ref[...] = ex
    wm_ref[...] = xs_ref[...] * jnp.dot(
        ex, exp_ref[...], preferred_element_type=jnp.float32)


def _messages(logit, m8, expand8, xrs):
    hc = xrs.shape[1]
    return pl.pallas_call(
        _msg_kernel,
        grid=(E_PAD // EB,),
        in_specs=[
            pl.BlockSpec((EB, H8), lambda i: (i, 0)),
            pl.BlockSpec((8, H8), lambda i: (0, 0)),
            pl.BlockSpec((H8, hc), lambda i: (0, 0)),
            pl.BlockSpec((EB, hc), lambda i: (i, 0)),
        ],
        out_specs=[
            pl.BlockSpec((EB, hc), lambda i: (i, 0)),
            pl.BlockSpec((EB, H8), lambda i: (i, 0)),
        ],
        out_shape=[
            jax.ShapeDtypeStruct((E_PAD, hc), jnp.float32),
            jax.ShapeDtypeStruct((E_PAD, H8), jnp.float32),
        ],
    )(logit, m8, expand8, xrs)


def _final_kernel(num_ref, den_ref, exp_ref, b_ref, out_ref, st_ref):
    pid = pl.program_id(0)
    den = jnp.dot(den_ref[...], exp_ref[...], preferred_element_type=jnp.float32)
    out = num_ref[...] / (den + 1e-16) + b_ref[0:1, :]
    out_ref[...] = out

    @pl.when(pid == 0)
    def _():
        st_ref[...] = jnp.zeros_like(st_ref)

    rows = lax.broadcasted_iota(jnp.int32, out.shape, 0) + pid * NB
    om = jnp.where(rows < N_REAL, out, 0.0)
    f = out.shape[1]
    part = jnp.concatenate(
        [jnp.sum(om, axis=0, keepdims=True),
         jnp.sum(om * om, axis=0, keepdims=True),
         jnp.zeros((6, f), jnp.float32)], axis=0)
    st_ref[...] += part


def _finalize(num, den, expand8, b8):
    f = num.shape[1]
    return pl.pallas_call(
        _final_kernel,
        grid=(N_PAD // NB,),
        in_specs=[
            pl.BlockSpec((NB, f), lambda i: (i, 0)),
            pl.BlockSpec((NB, H8), lambda i: (i, 0)),
            pl.BlockSpec((H8, f), lambda i: (0, 0)),
            pl.BlockSpec((8, f), lambda i: (0, 0)),
        ],
        out_specs=[
            pl.BlockSpec((NB, f), lambda i: (i, 0)),
            pl.BlockSpec((8, f), lambda i: (0, 0)),
        ],
        out_shape=[
            jax.ShapeDtypeStruct((N_PAD, f), jnp.float32),
            jax.ShapeDtypeStruct((8, f), jnp.float32),
        ],
    )(num, den, expand8, b8)


def _bn_kernel(out_ref, st_ref, g_ref, b_ref, h_ref):
    mu = st_ref[0:1, :] * (1.0 / N_REAL)
    var = st_ref[1:2, :] * (1.0 / N_REAL) - mu * mu
    y = (out_ref[...] - mu) * lax.rsqrt(var + 1e-5) * g_ref[0:1, :] + b_ref[0:1, :]
    h_ref[...] = jnp.where(y > 0, y, jnp.exp(jnp.minimum(y, 0.0)) - 1.0)


def _bn_apply(out, st, g8, b8):
    f = out.shape[1]
    return pl.pallas_call(
        _bn_kernel,
        grid=(N_PAD // NB,),
        in_specs=[
            pl.BlockSpec((NB, f), lambda i: (i, 0)),
            pl.BlockSpec((8, f), lambda i: (0, 0)),
            pl.BlockSpec((8, f), lambda i: (0, 0)),
            pl.BlockSpec((8, f), lambda i: (0, 0)),
        ],
        out_specs=pl.BlockSpec((NB, f), lambda i: (i, 0)),
        out_shape=jax.ShapeDtypeStruct((N_PAD, f), jnp.float32),
    )(out, st, g8, b8)


def _pool_kernel(nf_ref, b_ref, out_ref):
    pid = pl.program_id(0)

    @pl.when(pid == 0)
    def _():
        out_ref[...] = jnp.zeros_like(out_ref)

    bb = b_ref[...][:, 0:1]
    oh = (bb == lax.broadcasted_iota(jnp.int32, (NB, G), 1)).astype(jnp.float32)
    ext = jnp.concatenate(
        [nf_ref[...], jnp.ones((NB, 1), jnp.float32)], axis=1)
    out_ref[...] += lax.dot_general(
        oh, ext, (((0,), (0,)), ((), ())), preferred_element_type=jnp.float32)


def _pool(nf, batch8):
    return pl.pallas_call(
        _pool_kernel,
        grid=(N_PAD // NB,),
        in_specs=[
            pl.BlockSpec((NB, 128), lambda i: (i, 0)),
            pl.BlockSpec((NB, 8), lambda i: (i, 0)),
        ],
        out_specs=pl.BlockSpec((G, 129), lambda i: (0, 0)),
        out_shape=jax.ShapeDtypeStruct((G, 129), jnp.float32),
    )(nf, batch8)


def _heads_kernel(ge_ref, w1_ref, b1_ref, w2_ref, b2_ref, pw_ref, pb_ref,
                  tw_ref, tb_ref, sw_ref, sb_ref,
                  val_ref, pol_ref, tac_ref, str_ref):
    s = ge_ref[...]
    ge = s[:, :128] / jnp.maximum(s[:, 128:129], 1.0)
    vh = jnp.maximum(
        jnp.dot(ge, w1_ref[...], preferred_element_type=jnp.float32)
        + b1_ref[0:1, :], 0.0)
    val_ref[...] = jnp.tanh(
        jnp.dot(vh, w2_ref[...], preferred_element_type=jnp.float32)
        + b2_ref[0:1, :])
    pol_ref[...] = jnp.dot(
        ge, pw_ref[...], preferred_element_type=jnp.float32) + pb_ref[0:1, :]
    tac_ref[...] = jnp.dot(
        ge, tw_ref[...], preferred_element_type=jnp.float32) + tb_ref[0:1, :]
    str_ref[...] = jnp.dot(
        ge, sw_ref[...], preferred_element_type=jnp.float32) + sb_ref[0:1, :]


def _heads(ge, p):
    full = lambda a: pl.BlockSpec(a.shape, lambda: tuple(0 for _ in a.shape))
    args = [ge, p['v_W1'], _b8(p['v_b1']), p['v_W2'], _b8(p['v_b2']),
            p['p_W'], _b8(p['p_b']), p['t_W'], _b8(p['t_b']),
            p['s_W'], _b8(p['s_b'])]
    return pl.pallas_call(
        _heads_kernel,
        in_specs=[full(a) for a in args],
        out_specs=[
            pl.BlockSpec((G, 1), lambda: (0, 0)),
            pl.BlockSpec((G, 4096), lambda: (0, 0)),
            pl.BlockSpec((G, 1), lambda: (0, 0)),
            pl.BlockSpec((G, 1), lambda: (0, 0)),
        ],
        out_shape=[
            jax.ShapeDtypeStruct((G, 1), jnp.float32),
            jax.ShapeDtypeStruct((G, 4096), jnp.float32),
            jax.ShapeDtypeStruct((G, 1), jnp.float32),
            jax.ShapeDtypeStruct((G, 1), jnp.float32),
        ],
    )(*args)


def _b8(v):
    return jnp.broadcast_to(v[None, :], (8, v.shape[0]))


def _att_mats(att):
    heads, c = att.shape
    hc = heads * c
    hid = jnp.repeat(jnp.arange(heads), c)
    att_bd = jnp.zeros((hc, H8), jnp.float32).at[jnp.arange(hc), hid].set(
        att.reshape(-1))
    expand8 = jnp.zeros((H8, hc), jnp.float32).at[hid, jnp.arange(hc)].set(1.0)
    return att_bd, expand8


def _gat_layer(h, eap, srcp, dstp, wl, wr, we, att, b):
    xl, xr = _proj(h, wl, wr)
    xld = jnp.take(xl, dstp, axis=0)
    xrs = jnp.take(xr, srcp, axis=0)
    we8 = jnp.pad(we, ((0, 6), (0, 0)))
    att_bd, expand8 = _att_mats(att)
    logit = _logits(xld, xrs, eap, we8, att_bd)
    m8 = jnp.broadcast_to(jnp.max(logit, axis=0)[None, :], (8, H8))
    wmsg, exo = _messages(logit, m8, expand8, xrs)
    num = jax.ops.segment_sum(wmsg, dstp, num_segments=N_PAD)
    den = jax.ops.segment_sum(exo, dstp, num_segments=N_PAD)
    return _finalize(num, den, expand8, _b8(b))


def kernel(x, edge_index, edge_attr, batch, params):
    p = params
    src, dst = edge_index[0], edge_index[1]
    xp = jnp.pad(x, ((0, N_PAD - N_REAL), (0, 1)))
    eap = jnp.pad(edge_attr, ((0, E_PAD - E_REAL), (0, 6)))
    srcp = jnp.pad(src, (0, E_PAD - E_REAL))
    dstp = jnp.pad(dst, (0, E_PAD - E_REAL), constant_values=N_PAD - 1)
    batch8 = jnp.broadcast_to(
        jnp.pad(batch, (0, N_PAD - N_REAL), constant_values=G)[:, None],
        (N_PAD, 8))

    wl1 = jnp.pad(p['c1_Wl'], ((0, 1), (0, 0)))
    wr1 = jnp.pad(p['c1_Wr'], ((0, 1), (0, 0)))

    out, st = _gat_layer(xp, eap, srcp, dstp, wl1, wr1,
                         p['c1_We'], p['c1_att'], p['c1_b'])
    h = _bn_apply(out, st, _b8(p['n1_g']), _b8(p['n1_b']))
    out, st = _gat_layer(h, eap, srcp, dstp, p['c2_Wl'], p['c2_Wr'],
                         p['c2_We'], p['c2_att'], p['c2_b'])
    h = _bn_apply(out, st, _b8(p['n2_g']), _b8(p['n2_b']))
    out, st = _gat_layer(h, eap, srcp, dstp, p['c3_Wl'], p['c3_Wr'],
                         p['c3_We'], p['c3_att'], p['c3_b'])
    nf = _bn_apply(out, st, _b8(p['n3_g']), _b8(p['n3_b']))

    ge = _pool(nf, batch8)
    value, policy, tactic, strategic = _heads(ge, p)
    return (value, policy, tactic, strategic)
